# baseline (speedup 1.0000x reference)
import functools

import jax
import jax.numpy as jnp
from jax import lax
from jax.experimental import pallas as pl
from jax.experimental.pallas import tpu as pltpu
from jax.experimental.pallas import tpu_sc as plsc

_S = 512
_D = 128
_B = 20480
_C = 64
_NW = 32
_SEG_PER_W = _S // _NW
_NT = 40


def _gate_body(x_ref, b_ref, W_rat_ref, b_rat_ref, W_gate_ref, b_gate_ref,
               gate_ref, starts_ref, cnt_ref, *, nb):
    pid = pl.program_id(0)

    xb = x_ref[...].astype(jnp.bfloat16)
    feat = jnp.maximum(
        jnp.dot(xb, W_rat_ref[...].astype(jnp.bfloat16),
                preferred_element_type=jnp.float32)
        + b_rat_ref[...], 0.0)
    gpreT = lax.dot_general(W_gate_ref[...].astype(jnp.bfloat16),
                            feat.astype(jnp.bfloat16),
                            (((0,), (1,)), ((), ())),
                            preferred_element_type=jnp.float32)
    gate_ref[...] = jnp.reshape(jax.nn.sigmoid(gpreT + b_gate_ref[0, 0]),
                                (_B,))

    seg = b_ref[0]
    tgt = 16 * jax.lax.broadcasted_iota(jnp.int32, (_NT, 1), 0)
    partial = jnp.sum((seg < tgt).astype(jnp.float32), axis=1, keepdims=True)

    @pl.when(pid == 0)
    def _():
        cnt_ref[...] = jnp.zeros_like(cnt_ref)

    cnt_ref[...] += partial

    @pl.when(pid == nb - 1)
    def _():
        starts_ref[...] = jnp.broadcast_to(
            cnt_ref[...], (_NT, 16)).astype(jnp.int32)


def _compute_gate_and_starts(x, seg3d, W_rat, b_rat, W_gate, b_gate, nb):
    d = x.shape[1]
    return pl.pallas_call(
        functools.partial(_gate_body, nb=nb),
        grid=(nb,),
        in_specs=[
            pl.BlockSpec((_B, d), lambda i: (i, 0)),
            pl.BlockSpec((1, 1, _B), lambda i: (i, 0, 0)),
            pl.BlockSpec((d, d), lambda i: (0, 0)),
            pl.BlockSpec((1, d), lambda i: (0, 0)),
            pl.BlockSpec((d, 1), lambda i: (0, 0)),
            pl.BlockSpec((1, 1), lambda i: (0, 0)),
        ],
        out_specs=[
            pl.BlockSpec((_B,), lambda i: (i,)),
            pl.BlockSpec((_NT, 16), lambda i: (0, 0)),
        ],
        out_shape=[
            jax.ShapeDtypeStruct((nb * _B,), jnp.float32),
            jax.ShapeDtypeStruct((_NT, 16), jnp.int32),
        ],
        scratch_shapes=[pltpu.VMEM((_NT, 1), jnp.float32)],
    )(x, seg3d, W_rat, b_rat.reshape(1, d), W_gate, b_gate.reshape(1, 1))


def _sc_body(h_hbm, g_hbm, s_hbm, st_hbm,
             h_out, c_out, r_out, env_out,
             stbuf, hbuf, gbuf, sbuf, acc_gh, acc_h, racc2, cacc2,
             rst, est, cbuf, pbuf,
             hs0, gs0, ss0, hs1, gs1, ss1, *, n):
    cid = lax.axis_index("c")
    sid = lax.axis_index("s")
    wid = sid * 2 + cid
    seg0 = wid * _SEG_PER_W

    zero16 = jnp.zeros((16,), jnp.float32)

    def zero_row(j, _):
        for k in range(8):
            sl = pl.ds(16 * k, 16)
            acc_gh[j, sl] = zero16
            acc_h[j, sl] = zero16
        slz = pl.ds(0, 16)
        racc2[j, slz] = zero16
        cacc2[j, slz] = zero16
        cbuf[j, slz] = zero16
        return 0

    lax.fori_loop(0, _SEG_PER_W, zero_row, 0)
    cbuf[16, pl.ds(0, 16)] = zero16
    cbuf[17, pl.ds(0, 16)] = zero16
    pbuf[pl.ds(0, 16)] = lax.iota(jnp.int32, 16) * 0 - 1

    pltpu.sync_copy(st_hbm.at[pl.ds(16 * wid, 16)], stbuf.at[pl.ds(0, 16)])
    pltpu.sync_copy(st_hbm.at[pl.ds(16 * wid + 16, 16)],
                    stbuf.at[pl.ds(16, 16)])
    lo = stbuf[pl.ds(0, 16)][0]
    hi = stbuf[pl.ds(16, 16)][0]

    base0 = (lo // 8) * 8
    nch = (hi - base0 + _C - 1) // _C

    def cs_of(k):
        return jnp.minimum(base0 + k * _C, n - _C)

    def issue(k, b, hs, gs, ss):
        cs = cs_of(k)
        pltpu.async_copy(h_hbm.at[pl.ds(cs, _C), :],
                         hbuf.at[pl.ds(b * _C, _C)], hs)
        pltpu.async_copy(g_hbm.at[pl.ds(cs, _C)], gbuf.at[pl.ds(b * _C, _C)],
                         gs)
        pltpu.async_copy(s_hbm.at[pl.ds(cs, _C)], sbuf.at[pl.ds(b * _C, _C)],
                         ss)

    def wait_for(k, b, hs, gs, ss):
        cs = cs_of(k)
        pltpu.make_async_copy(h_hbm.at[pl.ds(cs, _C), :],
                              hbuf.at[pl.ds(b * _C, _C)], hs).wait()
        pltpu.make_async_copy(g_hbm.at[pl.ds(cs, _C)],
                              gbuf.at[pl.ds(b * _C, _C)], gs).wait()
        pltpu.make_async_copy(s_hbm.at[pl.ds(cs, _C)],
                              sbuf.at[pl.ds(b * _C, _C)], ss).wait()

    sems = ((hs0, gs0, ss0), (hs1, gs1, ss1))

    def flush(prev, cgh, ch, crv, ccv):
        loc = prev - seg0
        for k in range(8):
            sl = pl.ds(16 * k, 16)
            acc_gh[loc, sl] = acc_gh[loc, sl] + cgh[k]
            acc_h[loc, sl] = acc_h[loc, sl] + ch[k]
        sl16 = pl.ds(0, 16)
        racc2[loc, sl16] = racc2[loc, sl16] + crv
        cacc2[loc, sl16] = cacc2[loc, sl16] + ccv

    sl16 = pl.ds(0, 16)

    def load_state():
        cgh = [cbuf[k, sl16] for k in range(8)]
        ch = [cbuf[8 + k, sl16] for k in range(8)]
        return cgh, ch, cbuf[16, sl16], cbuf[17, sl16]

    def store_state(cgh, ch, crv, ccv):
        for k in range(8):
            cbuf[k, sl16] = cgh[k]
            cbuf[8 + k, sl16] = ch[k]
        cbuf[16, sl16] = crv
        cbuf[17, sl16] = ccv

    def make_group_body(b, cs, lo_k, hi_k):
        def group_body(j, _):
            off = b * _C + 16 * j
            row0 = cs + 16 * j
            sv_vec = sbuf[pl.ds(off, 16)]
            gv_vec = gbuf[pl.ds(off, 16)]
            prev0 = pbuf[sl16][0]
            fast = ((row0 >= lo_k) & (row0 + 16 <= hi_k)
                    & (sv_vec[0] == sv_vec[15]) & (sv_vec[0] == prev0))

            @pl.when(fast)
            def _():
                cgh, ch, crv, ccv = load_state()
                gl = [gv_vec[l] for l in range(16)]
                for k in range(8):
                    sl = pl.ds(16 * k, 16)
                    a = cgh[k]
                    hs = ch[k]
                    for l in range(16):
                        hv = hbuf[off + l, sl]
                        a = a + gl[l] * hv
                        hs = hs + hv
                    cgh[k] = a
                    ch[k] = hs
                store_state(cgh, ch, crv + gv_vec, ccv + 1.0)

            @pl.when(jnp.logical_not(fast))
            def _():
                prev = prev0
                cgh, ch, crv, ccv = load_state()
                lane = lax.iota(jnp.int32, 16)
                for l in range(16):
                    row = row0 + l
                    valid = (row >= lo_k) & (row < hi_k)
                    mf = jnp.where(valid, 1.0, 0.0)
                    sv = jnp.where(valid, sv_vec[l], prev)
                    geff = mf * gv_vec[l]
                    change = sv != prev

                    @pl.when(change & (prev >= 0))
                    def _(prev=prev, cgh=tuple(cgh), ch=tuple(ch),
                          crv=crv, ccv=ccv):
                        flush(prev, cgh, ch, crv, ccv)

                    keep = jnp.where(change, 0.0, 1.0)
                    onehot = jnp.where(lane == l, 1.0, 0.0)
                    for k in range(8):
                        hv = hbuf[off + l, pl.ds(16 * k, 16)]
                        cgh[k] = keep * cgh[k] + geff * hv
                        ch[k] = keep * ch[k] + mf * hv
                    crv = keep * crv + geff * onehot
                    ccv = keep * ccv + mf * onehot
                    prev = sv
                store_state(cgh, ch, crv, ccv)
                pbuf[sl16] = lane * 0 + prev
            return 0
        return group_body

    def pair_body(p, _):
        for b in (0, 1):
            k = 2 * p + b

            @pl.when(k + 1 < nch)
            def _():
                issue(k + 1, 1 - b, *sems[1 - b])

            @pl.when(k < nch)
            def _():
                wait_for(k, b, *sems[b])

            cs = cs_of(k)
            lo_k = jnp.maximum(lo, base0 + k * _C)
            hi_k = jnp.minimum(hi, base0 + k * _C + _C)
            ng = jnp.where(k < nch, _C // 16, 0)
            lax.fori_loop(0, ng, make_group_body(b, cs, lo_k, hi_k), 0)
        return 0

    @pl.when(nch > 0)
    def _():
        issue(0, 0, *sems[0])

    lax.fori_loop(0, (nch + 1) // 2, pair_body, 0)

    prev_fin = pbuf[sl16][0]

    @pl.when(prev_fin >= 0)
    def _():
        cgh, ch, crv, ccv = load_state()
        flush(prev_fin, cgh, ch, crv, ccv)

    def c_row(j, _):
        for k in range(8):
            sl = pl.ds(16 * k, 16)
            acc_h[j, sl] = acc_h[j, sl] - acc_gh[j, sl]
        return 0

    lax.fori_loop(0, _SEG_PER_W, c_row, 0)

    lane = lax.iota(jnp.int32, 16)
    rvec = zero16
    evec = zero16
    for j in range(_SEG_PER_W):
        v = racc2[j, pl.ds(0, 16)]
        w = cacc2[j, pl.ds(0, 16)]
        sr = v[0]
        sc = w[0]
        for l in range(1, 16):
            sr = sr + v[l]
            sc = sc + w[l]
        m = lane == j
        rvec = rvec + jnp.where(m, sr + 1e-8, 0.0)
        evec = evec + jnp.where(m, (sc - sr) + 1e-8, 0.0)
    rst[...] = rvec
    est[...] = evec

    pltpu.sync_copy(acc_gh, h_out.at[pl.ds(seg0, _SEG_PER_W), :])
    pltpu.sync_copy(acc_h, c_out.at[pl.ds(seg0, _SEG_PER_W), :])
    pltpu.sync_copy(rst, r_out.at[pl.ds(seg0, _SEG_PER_W)])
    pltpu.sync_copy(est, env_out.at[pl.ds(seg0, _SEG_PER_W)])


def _segment_reduce_sc(h_node, gate_flat, seg_flat, starts_flat, n):
    mesh = plsc.VectorSubcoreMesh(core_axis_name="c", subcore_axis_name="s")
    run = pl.kernel(
        functools.partial(_sc_body, n=n),
        mesh=mesh,
        out_type=[
            jax.ShapeDtypeStruct((_S, _D), jnp.float32),
            jax.ShapeDtypeStruct((_S, _D), jnp.float32),
            jax.ShapeDtypeStruct((_S,), jnp.float32),
            jax.ShapeDtypeStruct((_S,), jnp.float32),
        ],
        scratch_types=[
            pltpu.VMEM((32,), jnp.int32),
            pltpu.VMEM((2 * _C, _D), jnp.float32),
            pltpu.VMEM((2 * _C,), jnp.float32),
            pltpu.VMEM((2 * _C,), jnp.int32),
            pltpu.VMEM((_SEG_PER_W, _D), jnp.float32),
            pltpu.VMEM((_SEG_PER_W, _D), jnp.float32),
            pltpu.VMEM((_SEG_PER_W, 16), jnp.float32),
            pltpu.VMEM((_SEG_PER_W, 16), jnp.float32),
            pltpu.VMEM((16,), jnp.float32),
            pltpu.VMEM((16,), jnp.float32),
            pltpu.VMEM((18, 16), jnp.float32),
            pltpu.VMEM((16,), jnp.int32),
            pltpu.SemaphoreType.DMA,
            pltpu.SemaphoreType.DMA,
            pltpu.SemaphoreType.DMA,
            pltpu.SemaphoreType.DMA,
            pltpu.SemaphoreType.DMA,
            pltpu.SemaphoreType.DMA,
        ],
    )
    return run(h_node, gate_flat, seg_flat, starts_flat)


def kernel(x, h_node, batch, size, W_rat, b_rat, W_gate, b_gate):
    n, d = x.shape
    nb = (n + _B - 1) // _B
    npad = nb * _B

    seg = batch.astype(jnp.int32)
    seg3d = jnp.pad(seg, (0, npad - n), constant_values=_S).reshape(nb, 1, _B)

    gate_flat, starts2d = _compute_gate_and_starts(
        x, seg3d, W_rat, b_rat, W_gate, b_gate, nb)
    starts_flat = starts2d.reshape(_NT * 16)

    h_out, c_out, r_raw, env_raw = _segment_reduce_sc(
        h_node, gate_flat, seg, starts_flat, n)
    return (h_out, c_out, r_raw.reshape(_S, 1), env_raw.reshape(_S, 1))


# speedup vs baseline: 1.0680x; 1.0680x over previous
import functools

import jax
import jax.numpy as jnp
from jax import lax
from jax.experimental import pallas as pl
from jax.experimental.pallas import tpu as pltpu
from jax.experimental.pallas import tpu_sc as plsc

_S = 512
_D = 128
_B = 20480
_C = 192
_NW = 32
_SEG_PER_W = _S // _NW
_NT = 40


def _gate_body(x_ref, b_ref, W_rat_ref, b_rat_ref, W_gate_ref, b_gate_ref,
               gate_ref, starts_ref, cnt_ref, *, nb):
    pid = pl.program_id(0)

    xb = x_ref[...].astype(jnp.bfloat16)
    feat = jnp.maximum(
        jnp.dot(xb, W_rat_ref[...].astype(jnp.bfloat16),
                preferred_element_type=jnp.float32)
        + b_rat_ref[...], 0.0)
    gpreT = lax.dot_general(W_gate_ref[...].astype(jnp.bfloat16),
                            feat.astype(jnp.bfloat16),
                            (((0,), (1,)), ((), ())),
                            preferred_element_type=jnp.float32)
    gate_ref[...] = jnp.reshape(jax.nn.sigmoid(gpreT + b_gate_ref[0, 0]),
                                (_B,))

    seg = b_ref[0]
    tgt = 16 * jax.lax.broadcasted_iota(jnp.int32, (_NT, 1), 0)
    partial = jnp.sum((seg < tgt).astype(jnp.float32), axis=1, keepdims=True)

    @pl.when(pid == 0)
    def _():
        cnt_ref[...] = jnp.zeros_like(cnt_ref)

    cnt_ref[...] += partial

    @pl.when(pid == nb - 1)
    def _():
        starts_ref[...] = jnp.broadcast_to(
            cnt_ref[...], (_NT, 16)).astype(jnp.int32)


def _compute_gate_and_starts(x, seg3d, W_rat, b_rat, W_gate, b_gate, nb):
    d = x.shape[1]
    return pl.pallas_call(
        functools.partial(_gate_body, nb=nb),
        grid=(nb,),
        in_specs=[
            pl.BlockSpec((_B, d), lambda i: (i, 0)),
            pl.BlockSpec((1, 1, _B), lambda i: (i, 0, 0)),
            pl.BlockSpec((d, d), lambda i: (0, 0)),
            pl.BlockSpec((1, d), lambda i: (0, 0)),
            pl.BlockSpec((d, 1), lambda i: (0, 0)),
            pl.BlockSpec((1, 1), lambda i: (0, 0)),
        ],
        out_specs=[
            pl.BlockSpec((_B,), lambda i: (i,)),
            pl.BlockSpec((_NT, 16), lambda i: (0, 0)),
        ],
        out_shape=[
            jax.ShapeDtypeStruct((nb * _B,), jnp.float32),
            jax.ShapeDtypeStruct((_NT, 16), jnp.int32),
        ],
        scratch_shapes=[pltpu.VMEM((_NT, 1), jnp.float32)],
    )(x, seg3d, W_rat, b_rat.reshape(1, d), W_gate, b_gate.reshape(1, 1))


def _sc_body(h_hbm, g_hbm, s_hbm, st_hbm,
             h_out, c_out, r_out, env_out,
             stbuf, hbuf, gbuf, sbuf, acc_gh, acc_h, racc2, cacc2,
             rst, est, cbuf, pbuf,
             hs0, gs0, ss0, hs1, gs1, ss1, *, n):
    cid = lax.axis_index("c")
    sid = lax.axis_index("s")
    wid = sid * 2 + cid
    seg0 = wid * _SEG_PER_W

    zero16 = jnp.zeros((16,), jnp.float32)

    def zero_row(j, _):
        for k in range(8):
            sl = pl.ds(16 * k, 16)
            acc_gh[j, sl] = zero16
            acc_h[j, sl] = zero16
        slz = pl.ds(0, 16)
        racc2[j, slz] = zero16
        cacc2[j, slz] = zero16
        cbuf[j, slz] = zero16
        return 0

    lax.fori_loop(0, _SEG_PER_W, zero_row, 0)
    cbuf[16, pl.ds(0, 16)] = zero16
    cbuf[17, pl.ds(0, 16)] = zero16
    pbuf[pl.ds(0, 16)] = lax.iota(jnp.int32, 16) * 0 - 1

    pltpu.sync_copy(st_hbm.at[pl.ds(16 * wid, 16)], stbuf.at[pl.ds(0, 16)])
    pltpu.sync_copy(st_hbm.at[pl.ds(16 * wid + 16, 16)],
                    stbuf.at[pl.ds(16, 16)])
    lo = stbuf[pl.ds(0, 16)][0]
    hi = stbuf[pl.ds(16, 16)][0]

    base0 = (lo // 8) * 8
    nch = (hi - base0 + _C - 1) // _C

    def cs_of(k):
        return jnp.minimum(base0 + k * _C, n - _C)

    def issue(k, b, hs, gs, ss):
        cs = cs_of(k)
        pltpu.async_copy(h_hbm.at[pl.ds(cs, _C), :],
                         hbuf.at[pl.ds(b * _C, _C)], hs)
        pltpu.async_copy(g_hbm.at[pl.ds(cs, _C)], gbuf.at[pl.ds(b * _C, _C)],
                         gs)
        pltpu.async_copy(s_hbm.at[pl.ds(cs, _C)], sbuf.at[pl.ds(b * _C, _C)],
                         ss)

    def wait_for(k, b, hs, gs, ss):
        cs = cs_of(k)
        pltpu.make_async_copy(h_hbm.at[pl.ds(cs, _C), :],
                              hbuf.at[pl.ds(b * _C, _C)], hs).wait()
        pltpu.make_async_copy(g_hbm.at[pl.ds(cs, _C)],
                              gbuf.at[pl.ds(b * _C, _C)], gs).wait()
        pltpu.make_async_copy(s_hbm.at[pl.ds(cs, _C)],
                              sbuf.at[pl.ds(b * _C, _C)], ss).wait()

    sems = ((hs0, gs0, ss0), (hs1, gs1, ss1))

    def flush(prev, cgh, ch, crv, ccv):
        loc = prev - seg0
        for k in range(8):
            sl = pl.ds(16 * k, 16)
            acc_gh[loc, sl] = acc_gh[loc, sl] + cgh[k]
            acc_h[loc, sl] = acc_h[loc, sl] + ch[k]
        sl16 = pl.ds(0, 16)
        racc2[loc, sl16] = racc2[loc, sl16] + crv
        cacc2[loc, sl16] = cacc2[loc, sl16] + ccv

    sl16 = pl.ds(0, 16)

    def load_state():
        cgh = [cbuf[k, sl16] for k in range(8)]
        ch = [cbuf[8 + k, sl16] for k in range(8)]
        return cgh, ch, cbuf[16, sl16], cbuf[17, sl16]

    def store_state(cgh, ch, crv, ccv):
        for k in range(8):
            cbuf[k, sl16] = cgh[k]
            cbuf[8 + k, sl16] = ch[k]
        cbuf[16, sl16] = crv
        cbuf[17, sl16] = ccv

    def make_group_body(b, cs, lo_k, hi_k):
        def group_body(j, _):
            off = b * _C + 16 * j
            row0 = cs + 16 * j
            sv_vec = sbuf[pl.ds(off, 16)]
            gv_vec = gbuf[pl.ds(off, 16)]
            prev0 = pbuf[sl16][0]
            fast = ((row0 >= lo_k) & (row0 + 16 <= hi_k)
                    & (sv_vec[0] == sv_vec[15]) & (sv_vec[0] == prev0))

            @pl.when(fast)
            def _():
                cgh, ch, crv, ccv = load_state()
                gl = [gv_vec[l] for l in range(16)]
                for k in range(8):
                    sl = pl.ds(16 * k, 16)
                    a = cgh[k]
                    hs = ch[k]
                    for l in range(16):
                        hv = hbuf[off + l, sl]
                        a = a + gl[l] * hv
                        hs = hs + hv
                    cgh[k] = a
                    ch[k] = hs
                store_state(cgh, ch, crv + gv_vec, ccv + 1.0)

            @pl.when(jnp.logical_not(fast))
            def _():
                prev = prev0
                cgh, ch, crv, ccv = load_state()
                lane = lax.iota(jnp.int32, 16)
                for l in range(16):
                    row = row0 + l
                    valid = (row >= lo_k) & (row < hi_k)
                    mf = jnp.where(valid, 1.0, 0.0)
                    sv = jnp.where(valid, sv_vec[l], prev)
                    geff = mf * gv_vec[l]
                    change = sv != prev

                    @pl.when(change & (prev >= 0))
                    def _(prev=prev, cgh=tuple(cgh), ch=tuple(ch),
                          crv=crv, ccv=ccv):
                        flush(prev, cgh, ch, crv, ccv)

                    keep = jnp.where(change, 0.0, 1.0)
                    onehot = jnp.where(lane == l, 1.0, 0.0)
                    for k in range(8):
                        hv = hbuf[off + l, pl.ds(16 * k, 16)]
                        cgh[k] = keep * cgh[k] + geff * hv
                        ch[k] = keep * ch[k] + mf * hv
                    crv = keep * crv + geff * onehot
                    ccv = keep * ccv + mf * onehot
                    prev = sv
                store_state(cgh, ch, crv, ccv)
                pbuf[sl16] = lane * 0 + prev
            return 0
        return group_body

    def pair_body(p, _):
        for b in (0, 1):
            k = 2 * p + b

            @pl.when(k + 1 < nch)
            def _():
                issue(k + 1, 1 - b, *sems[1 - b])

            @pl.when(k < nch)
            def _():
                wait_for(k, b, *sems[b])

            cs = cs_of(k)
            lo_k = jnp.maximum(lo, base0 + k * _C)
            hi_k = jnp.minimum(hi, base0 + k * _C + _C)
            ng = jnp.where(k < nch, _C // 16, 0)
            lax.fori_loop(0, ng, make_group_body(b, cs, lo_k, hi_k), 0)
        return 0

    @pl.when(nch > 0)
    def _():
        issue(0, 0, *sems[0])

    lax.fori_loop(0, (nch + 1) // 2, pair_body, 0)

    prev_fin = pbuf[sl16][0]

    @pl.when(prev_fin >= 0)
    def _():
        cgh, ch, crv, ccv = load_state()
        flush(prev_fin, cgh, ch, crv, ccv)

    def c_row(j, _):
        for k in range(8):
            sl = pl.ds(16 * k, 16)
            acc_h[j, sl] = acc_h[j, sl] - acc_gh[j, sl]
        return 0

    lax.fori_loop(0, _SEG_PER_W, c_row, 0)

    lane = lax.iota(jnp.int32, 16)
    rvec = zero16
    evec = zero16
    for j in range(_SEG_PER_W):
        v = racc2[j, pl.ds(0, 16)]
        w = cacc2[j, pl.ds(0, 16)]
        sr = v[0]
        sc = w[0]
        for l in range(1, 16):
            sr = sr + v[l]
            sc = sc + w[l]
        m = lane == j
        rvec = rvec + jnp.where(m, sr + 1e-8, 0.0)
        evec = evec + jnp.where(m, (sc - sr) + 1e-8, 0.0)
    rst[...] = rvec
    est[...] = evec

    pltpu.sync_copy(acc_gh, h_out.at[pl.ds(seg0, _SEG_PER_W), :])
    pltpu.sync_copy(acc_h, c_out.at[pl.ds(seg0, _SEG_PER_W), :])
    pltpu.sync_copy(rst, r_out.at[pl.ds(seg0, _SEG_PER_W)])
    pltpu.sync_copy(est, env_out.at[pl.ds(seg0, _SEG_PER_W)])


def _segment_reduce_sc(h_node, gate_flat, seg_flat, starts_flat, n):
    mesh = plsc.VectorSubcoreMesh(core_axis_name="c", subcore_axis_name="s")
    run = pl.kernel(
        functools.partial(_sc_body, n=n),
        mesh=mesh,
        out_type=[
            jax.ShapeDtypeStruct((_S, _D), jnp.float32),
            jax.ShapeDtypeStruct((_S, _D), jnp.float32),
            jax.ShapeDtypeStruct((_S,), jnp.float32),
            jax.ShapeDtypeStruct((_S,), jnp.float32),
        ],
        scratch_types=[
            pltpu.VMEM((32,), jnp.int32),
            pltpu.VMEM((2 * _C, _D), jnp.float32),
            pltpu.VMEM((2 * _C,), jnp.float32),
            pltpu.VMEM((2 * _C,), jnp.int32),
            pltpu.VMEM((_SEG_PER_W, _D), jnp.float32),
            pltpu.VMEM((_SEG_PER_W, _D), jnp.float32),
            pltpu.VMEM((_SEG_PER_W, 16), jnp.float32),
            pltpu.VMEM((_SEG_PER_W, 16), jnp.float32),
            pltpu.VMEM((16,), jnp.float32),
            pltpu.VMEM((16,), jnp.float32),
            pltpu.VMEM((18, 16), jnp.float32),
            pltpu.VMEM((16,), jnp.int32),
            pltpu.SemaphoreType.DMA,
            pltpu.SemaphoreType.DMA,
            pltpu.SemaphoreType.DMA,
            pltpu.SemaphoreType.DMA,
            pltpu.SemaphoreType.DMA,
            pltpu.SemaphoreType.DMA,
        ],
    )
    return run(h_node, gate_flat, seg_flat, starts_flat)


def kernel(x, h_node, batch, size, W_rat, b_rat, W_gate, b_gate):
    n, d = x.shape
    nb = (n + _B - 1) // _B
    npad = nb * _B

    seg = batch.astype(jnp.int32)
    seg3d = jnp.pad(seg, (0, npad - n), constant_values=_S).reshape(nb, 1, _B)

    gate_flat, starts2d = _compute_gate_and_starts(
        x, seg3d, W_rat, b_rat, W_gate, b_gate, nb)
    starts_flat = starts2d.reshape(_NT * 16)

    h_out, c_out, r_raw, env_raw = _segment_reduce_sc(
        h_node, gate_flat, seg, starts_flat, n)
    return (h_out, c_out, r_raw.reshape(_S, 1), env_raw.reshape(_S, 1))


# speedup vs baseline: 1.0721x; 1.0039x over previous
import functools

import jax
import jax.numpy as jnp
from jax import lax
from jax.experimental import pallas as pl
from jax.experimental.pallas import tpu as pltpu
from jax.experimental.pallas import tpu_sc as plsc

_S = 512
_D = 128
_B = 20480
_C = 128
_NW = 32
_SEG_PER_W = _S // _NW
_NT = 40


def _gate_body(x_ref, b_ref, W_rat_ref, b_rat_ref, W_gate_ref, b_gate_ref,
               gate_ref, starts_ref, cnt_ref, *, nb):
    pid = pl.program_id(0)

    xb = x_ref[...].astype(jnp.bfloat16)
    feat = jnp.maximum(
        jnp.dot(xb, W_rat_ref[...].astype(jnp.bfloat16),
                preferred_element_type=jnp.float32)
        + b_rat_ref[...], 0.0)
    gpreT = lax.dot_general(W_gate_ref[...].astype(jnp.bfloat16),
                            feat.astype(jnp.bfloat16),
                            (((0,), (1,)), ((), ())),
                            preferred_element_type=jnp.float32)
    gate_ref[...] = jnp.reshape(jax.nn.sigmoid(gpreT + b_gate_ref[0, 0]),
                                (_B,))

    seg = b_ref[0]
    tgt = 16 * jax.lax.broadcasted_iota(jnp.int32, (_NT, 1), 0)
    partial = jnp.sum((seg < tgt).astype(jnp.float32), axis=1, keepdims=True)

    @pl.when(pid == 0)
    def _():
        cnt_ref[...] = jnp.zeros_like(cnt_ref)

    cnt_ref[...] += partial

    @pl.when(pid == nb - 1)
    def _():
        starts_ref[...] = jnp.broadcast_to(
            cnt_ref[...], (_NT, 16)).astype(jnp.int32)


def _compute_gate_and_starts(x, seg3d, W_rat, b_rat, W_gate, b_gate, nb):
    d = x.shape[1]
    return pl.pallas_call(
        functools.partial(_gate_body, nb=nb),
        grid=(nb,),
        in_specs=[
            pl.BlockSpec((_B, d), lambda i: (i, 0)),
            pl.BlockSpec((1, 1, _B), lambda i: (i, 0, 0)),
            pl.BlockSpec((d, d), lambda i: (0, 0)),
            pl.BlockSpec((1, d), lambda i: (0, 0)),
            pl.BlockSpec((d, 1), lambda i: (0, 0)),
            pl.BlockSpec((1, 1), lambda i: (0, 0)),
        ],
        out_specs=[
            pl.BlockSpec((_B,), lambda i: (i,)),
            pl.BlockSpec((_NT, 16), lambda i: (0, 0)),
        ],
        out_shape=[
            jax.ShapeDtypeStruct((nb * _B,), jnp.float32),
            jax.ShapeDtypeStruct((_NT, 16), jnp.int32),
        ],
        scratch_shapes=[pltpu.VMEM((_NT, 1), jnp.float32)],
    )(x, seg3d, W_rat, b_rat.reshape(1, d), W_gate, b_gate.reshape(1, 1))


def _sc_body(h_hbm, g_hbm, s_hbm, st_hbm,
             h_out, c_out, r_out, env_out,
             stbuf, hbuf, gbuf, sbuf, acc_gh, acc_h, racc2, cacc2,
             rst, est, cbuf, pbuf,
             hs0, gs0, ss0, hs1, gs1, ss1, *, n):
    cid = lax.axis_index("c")
    sid = lax.axis_index("s")
    wid = sid * 2 + cid
    seg0 = wid * _SEG_PER_W

    zero16 = jnp.zeros((16,), jnp.float32)

    def zero_row(j, _):
        for k in range(8):
            sl = pl.ds(16 * k, 16)
            acc_gh[j, sl] = zero16
            acc_h[j, sl] = zero16
        slz = pl.ds(0, 16)
        racc2[j, slz] = zero16
        cacc2[j, slz] = zero16
        cbuf[j, slz] = zero16
        return 0

    lax.fori_loop(0, _SEG_PER_W, zero_row, 0)
    cbuf[16, pl.ds(0, 16)] = zero16
    cbuf[17, pl.ds(0, 16)] = zero16
    pbuf[pl.ds(0, 16)] = lax.iota(jnp.int32, 16) * 0 - 1

    pltpu.sync_copy(st_hbm.at[pl.ds(16 * wid, 16)], stbuf.at[pl.ds(0, 16)])
    pltpu.sync_copy(st_hbm.at[pl.ds(16 * wid + 16, 16)],
                    stbuf.at[pl.ds(16, 16)])
    lo = stbuf[pl.ds(0, 16)][0]
    hi = stbuf[pl.ds(16, 16)][0]

    base0 = (lo // 8) * 8
    nch = (hi - base0 + _C - 1) // _C

    def cs_of(k):
        return jnp.minimum(base0 + k * _C, n - _C)

    def issue(k, b, hs, gs, ss):
        cs = cs_of(k)
        pltpu.async_copy(h_hbm.at[pl.ds(cs, _C), :],
                         hbuf.at[pl.ds(b * _C, _C)], hs)
        pltpu.async_copy(g_hbm.at[pl.ds(cs, _C)], gbuf.at[pl.ds(b * _C, _C)],
                         gs)
        pltpu.async_copy(s_hbm.at[pl.ds(cs, _C)], sbuf.at[pl.ds(b * _C, _C)],
                         ss)

    def wait_for(k, b, hs, gs, ss):
        cs = cs_of(k)
        pltpu.make_async_copy(h_hbm.at[pl.ds(cs, _C), :],
                              hbuf.at[pl.ds(b * _C, _C)], hs).wait()
        pltpu.make_async_copy(g_hbm.at[pl.ds(cs, _C)],
                              gbuf.at[pl.ds(b * _C, _C)], gs).wait()
        pltpu.make_async_copy(s_hbm.at[pl.ds(cs, _C)],
                              sbuf.at[pl.ds(b * _C, _C)], ss).wait()

    sems = ((hs0, gs0, ss0), (hs1, gs1, ss1))

    def flush(prev, cgh, ch, crv, ccv):
        loc = prev - seg0
        for k in range(8):
            sl = pl.ds(16 * k, 16)
            acc_gh[loc, sl] = acc_gh[loc, sl] + cgh[k]
            acc_h[loc, sl] = acc_h[loc, sl] + ch[k]
        sl16 = pl.ds(0, 16)
        racc2[loc, sl16] = racc2[loc, sl16] + crv
        cacc2[loc, sl16] = cacc2[loc, sl16] + ccv

    sl16 = pl.ds(0, 16)

    def load_state():
        cgh = [cbuf[k, sl16] for k in range(8)]
        ch = [cbuf[8 + k, sl16] for k in range(8)]
        return cgh, ch, cbuf[16, sl16], cbuf[17, sl16]

    def store_state(cgh, ch, crv, ccv):
        for k in range(8):
            cbuf[k, sl16] = cgh[k]
            cbuf[8 + k, sl16] = ch[k]
        cbuf[16, sl16] = crv
        cbuf[17, sl16] = ccv

    def make_group_body(b, cs, lo_k, hi_k):
        def group_body(j, _):
            off = b * _C + 16 * j
            row0 = cs + 16 * j
            sv_vec = sbuf[pl.ds(off, 16)]
            gv_vec = gbuf[pl.ds(off, 16)]
            prev0 = pbuf[sl16][0]
            fast = ((row0 >= lo_k) & (row0 + 16 <= hi_k)
                    & (sv_vec[0] == sv_vec[15]) & (sv_vec[0] == prev0))

            @pl.when(fast)
            def _():
                cgh, ch, crv, ccv = load_state()
                gl = [gv_vec[l] for l in range(16)]
                for k in range(8):
                    sl = pl.ds(16 * k, 16)
                    a = cgh[k]
                    hs = ch[k]
                    for l in range(16):
                        hv = hbuf[off + l, sl]
                        a = a + gl[l] * hv
                        hs = hs + hv
                    cgh[k] = a
                    ch[k] = hs
                store_state(cgh, ch, crv + gv_vec, ccv + 1.0)

            @pl.when(jnp.logical_not(fast))
            def _():
                prev = prev0
                cgh, ch, crv, ccv = load_state()
                lane = lax.iota(jnp.int32, 16)
                for l in range(16):
                    row = row0 + l
                    valid = (row >= lo_k) & (row < hi_k)
                    mf = jnp.where(valid, 1.0, 0.0)
                    sv = jnp.where(valid, sv_vec[l], prev)
                    geff = mf * gv_vec[l]
                    change = sv != prev

                    @pl.when(change & (prev >= 0))
                    def _(prev=prev, cgh=tuple(cgh), ch=tuple(ch),
                          crv=crv, ccv=ccv):
                        flush(prev, cgh, ch, crv, ccv)

                    keep = jnp.where(change, 0.0, 1.0)
                    onehot = jnp.where(lane == l, 1.0, 0.0)
                    for k in range(8):
                        hv = hbuf[off + l, pl.ds(16 * k, 16)]
                        cgh[k] = keep * cgh[k] + geff * hv
                        ch[k] = keep * ch[k] + mf * hv
                    crv = keep * crv + geff * onehot
                    ccv = keep * ccv + mf * onehot
                    prev = sv
                store_state(cgh, ch, crv, ccv)
                pbuf[sl16] = lane * 0 + prev
            return 0
        return group_body

    def pair_body(p, _):
        for b in (0, 1):
            k = 2 * p + b

            @pl.when(k + 1 < nch)
            def _():
                issue(k + 1, 1 - b, *sems[1 - b])

            @pl.when(k < nch)
            def _():
                wait_for(k, b, *sems[b])

            cs = cs_of(k)
            lo_k = jnp.maximum(lo, base0 + k * _C)
            hi_k = jnp.minimum(hi, base0 + k * _C + _C)
            ng = jnp.where(k < nch, _C // 16, 0)
            lax.fori_loop(0, ng, make_group_body(b, cs, lo_k, hi_k), 0)
        return 0

    @pl.when(nch > 0)
    def _():
        issue(0, 0, *sems[0])

    lax.fori_loop(0, (nch + 1) // 2, pair_body, 0)

    prev_fin = pbuf[sl16][0]

    @pl.when(prev_fin >= 0)
    def _():
        cgh, ch, crv, ccv = load_state()
        flush(prev_fin, cgh, ch, crv, ccv)

    def c_row(j, _):
        for k in range(8):
            sl = pl.ds(16 * k, 16)
            acc_h[j, sl] = acc_h[j, sl] - acc_gh[j, sl]
        return 0

    lax.fori_loop(0, _SEG_PER_W, c_row, 0)

    lane = lax.iota(jnp.int32, 16)
    rvec = zero16
    evec = zero16
    for j in range(_SEG_PER_W):
        v = racc2[j, pl.ds(0, 16)]
        w = cacc2[j, pl.ds(0, 16)]
        sr = v[0]
        sc = w[0]
        for l in range(1, 16):
            sr = sr + v[l]
            sc = sc + w[l]
        m = lane == j
        rvec = rvec + jnp.where(m, sr + 1e-8, 0.0)
        evec = evec + jnp.where(m, (sc - sr) + 1e-8, 0.0)
    rst[...] = rvec
    est[...] = evec

    pltpu.sync_copy(acc_gh, h_out.at[pl.ds(seg0, _SEG_PER_W), :])
    pltpu.sync_copy(acc_h, c_out.at[pl.ds(seg0, _SEG_PER_W), :])
    pltpu.sync_copy(rst, r_out.at[pl.ds(seg0, _SEG_PER_W)])
    pltpu.sync_copy(est, env_out.at[pl.ds(seg0, _SEG_PER_W)])


def _segment_reduce_sc(h_node, gate_flat, seg_flat, starts_flat, n):
    mesh = plsc.VectorSubcoreMesh(core_axis_name="c", subcore_axis_name="s")
    run = pl.kernel(
        functools.partial(_sc_body, n=n),
        mesh=mesh,
        out_type=[
            jax.ShapeDtypeStruct((_S, _D), jnp.float32),
            jax.ShapeDtypeStruct((_S, _D), jnp.float32),
            jax.ShapeDtypeStruct((_S,), jnp.float32),
            jax.ShapeDtypeStruct((_S,), jnp.float32),
        ],
        scratch_types=[
            pltpu.VMEM((32,), jnp.int32),
            pltpu.VMEM((2 * _C, _D), jnp.float32),
            pltpu.VMEM((2 * _C,), jnp.float32),
            pltpu.VMEM((2 * _C,), jnp.int32),
            pltpu.VMEM((_SEG_PER_W, _D), jnp.float32),
            pltpu.VMEM((_SEG_PER_W, _D), jnp.float32),
            pltpu.VMEM((_SEG_PER_W, 16), jnp.float32),
            pltpu.VMEM((_SEG_PER_W, 16), jnp.float32),
            pltpu.VMEM((16,), jnp.float32),
            pltpu.VMEM((16,), jnp.float32),
            pltpu.VMEM((18, 16), jnp.float32),
            pltpu.VMEM((16,), jnp.int32),
            pltpu.SemaphoreType.DMA,
            pltpu.SemaphoreType.DMA,
            pltpu.SemaphoreType.DMA,
            pltpu.SemaphoreType.DMA,
            pltpu.SemaphoreType.DMA,
            pltpu.SemaphoreType.DMA,
        ],
    )
    return run(h_node, gate_flat, seg_flat, starts_flat)


def kernel(x, h_node, batch, size, W_rat, b_rat, W_gate, b_gate):
    n, d = x.shape
    nb = (n + _B - 1) // _B
    npad = nb * _B

    seg = batch.astype(jnp.int32)
    seg3d = jnp.pad(seg, (0, npad - n), constant_values=_S).reshape(nb, 1, _B)

    gate_flat, starts2d = _compute_gate_and_starts(
        x, seg3d, W_rat, b_rat, W_gate, b_gate, nb)
    starts_flat = starts2d.reshape(_NT * 16)

    h_out, c_out, r_raw, env_raw = _segment_reduce_sc(
        h_node, gate_flat, seg, starts_flat, n)
    return (h_out, c_out, r_raw.reshape(_S, 1), env_raw.reshape(_S, 1))
